# baseline (speedup 1.0000x reference)
import jax
import jax.numpy as jnp
from jax import lax
from jax.experimental import pallas as pl
from jax.experimental.pallas import tpu as pltpu
from jax.experimental.pallas import tpu_sc as plsc

N_EDGES = 320000
D = 128
N_SEG = 5
NC, NS = 2, 16
NW = NC * NS
CHUNK = 40
CPW = N_EDGES // CHUNK // NW
EPW = CHUNK * CPW
IBLK = 50
IB_EDGES = IBLK * CHUNK
NBUF = 4


def _body(intra_h, node_h, src_h, dst_h, et_h, rel_h, out_h,
          src_v, dst_v, et_v, asm0, asm1, asm2, asm3,
          g0, g1, g2, g3, w0, w1, w2, w3):
    wid = lax.axis_index("s") * NC + lax.axis_index("c")
    e0 = wid * EPW
    asms = (asm0, asm1, asm2, asm3)
    gsems = (g0, g1, g2, g3)
    wsems = (w0, w1, w2, w3)

    def stage_block(j):
        off = e0 + (j // IBLK) * IB_EDGES
        pltpu.sync_copy(src_h.at[pl.ds(off, IB_EDGES)], src_v)
        pltpu.sync_copy(dst_h.at[pl.ds(off, IB_EDGES)], dst_v)
        pltpu.sync_copy(et_h.at[pl.ds(off, IB_EDGES)], et_v)

    def fire_gathers(j, b):
        ioff = (j % IBLK) * CHUNK
        a = asms[b]
        si = src_v.at[pl.ds(ioff, CHUNK)]
        di = dst_v.at[pl.ds(ioff, CHUNK)]
        ti = et_v.at[pl.ds(ioff, CHUNK)]
        pltpu.async_copy(intra_h.at[si], a.at[:, pl.ds(0 * D, D)], gsems[b])
        pltpu.async_copy(intra_h.at[di], a.at[:, pl.ds(1 * D, D)], gsems[b])
        pltpu.async_copy(node_h.at[si], a.at[:, pl.ds(2 * D, D)], gsems[b])
        pltpu.async_copy(node_h.at[di], a.at[:, pl.ds(3 * D, D)], gsems[b])
        pltpu.async_copy(rel_h.at[ti], a.at[:, pl.ds(4 * D, D)], gsems[b])

    def drain_gathers(b):
        pltpu.make_async_copy(out_h.at[pl.ds(0, CHUNK)], asms[b], gsems[b]).wait()

    def reclaim_write(b):
        pltpu.make_async_copy(asms[b], out_h.at[pl.ds(0, CHUNK)], wsems[b]).wait()

    stage_block(0)
    fire_gathers(0, 0)

    def outer(i, carry):
        for b in range(NBUF):
            j = NBUF * i + b

            @pl.when(j < CPW)
            def _():
                drain_gathers(b)
                pltpu.async_copy(asms[b], out_h.at[pl.ds(e0 + j * CHUNK, CHUNK)],
                                 wsems[b])

                @pl.when(j + 1 < CPW)
                def _():
                    bn = (b + 1) % NBUF

                    @pl.when((j + 1) % IBLK == 0)
                    def _():
                        stage_block(j + 1)

                    @pl.when(j >= 3)
                    def _():
                        reclaim_write(bn)

                    fire_gathers(j + 1, bn)

        return carry

    lax.fori_loop(0, (CPW + NBUF - 1) // NBUF, outer, None)
    for b in range(NBUF):
        reclaim_write(b)


_gather_concat = pl.kernel(
    _body,
    out_type=jax.ShapeDtypeStruct((N_EDGES, N_SEG * D), jnp.float32),
    mesh=plsc.VectorSubcoreMesh(core_axis_name="c", subcore_axis_name="s"),
    scratch_types=[
        pltpu.VMEM((IB_EDGES,), jnp.int32),
        pltpu.VMEM((IB_EDGES,), jnp.int32),
        pltpu.VMEM((IB_EDGES,), jnp.int32),
        pltpu.VMEM((CHUNK, N_SEG * D), jnp.float32),
        pltpu.VMEM((CHUNK, N_SEG * D), jnp.float32),
        pltpu.VMEM((CHUNK, N_SEG * D), jnp.float32),
        pltpu.VMEM((CHUNK, N_SEG * D), jnp.float32),
        pltpu.SemaphoreType.DMA,
        pltpu.SemaphoreType.DMA,
        pltpu.SemaphoreType.DMA,
        pltpu.SemaphoreType.DMA,
        pltpu.SemaphoreType.DMA,
        pltpu.SemaphoreType.DMA,
        pltpu.SemaphoreType.DMA,
        pltpu.SemaphoreType.DMA,
    ],
)


@jax.jit
def kernel(intra, node_repr, edge_index, edge_type, rel_emb):
    src = edge_index[0].astype(jnp.int32)
    dst = edge_index[1].astype(jnp.int32)
    et = edge_type.astype(jnp.int32)
    return _gather_concat(intra, node_repr, src, dst, et, rel_emb)
